# baseline (speedup 1.0000x reference)
import functools

import jax
import jax.numpy as jnp
from jax import lax
from jax.experimental import pallas as pl
from jax.experimental.pallas import tpu as pltpu
from jax.experimental.pallas import tpu_sc as plsc

NC = 2
NS = 16
NW = NC * NS
LANES = 16
C = 112
NBUF = 3
IBUF = 6
PERIOD = 6
SPLIT_FRAC0 = 0.9


def _make_agg(npad, d, nch0, nch1):
  rows_per_tile = npad // NS
  full, rem = divmod(rows_per_tile, C)
  assert nch0 % PERIOD == 0 and nch1 % PERIOD == 0
  mesh = plsc.VectorSubcoreMesh(core_axis_name="c", subcore_axis_name="s")

  @functools.partial(
      pl.kernel,
      out_type=jax.ShapeDtypeStruct((NC, npad, d), jnp.float32),
      mesh=mesh,
      scratch_types=[
          pltpu.VMEM((IBUF, 2, C), jnp.int32),
          pltpu.VMEM((IBUF, 1, C), jnp.float32),
          pltpu.VMEM((NBUF, C, d), jnp.float32),
          pltpu.VMEM_SHARED((npad, d), jnp.float32),
          pltpu.SemaphoreType.DMA((IBUF,)),
          pltpu.SemaphoreType.DMA((IBUF,)),
          pltpu.SemaphoreType.DMA((NBUF,)),
          pltpu.SemaphoreType.DMA((NBUF,)),
      ],
  )
  def agg(table_hbm, eidx_hbm, ew_hbm, out_hbm,
          iring, wring, rowbuf, acc, isem, wsem, gsem, ssem):
    c = lax.axis_index("c")
    s = lax.axis_index("s")
    nch = jnp.where(c == 0, nch0, nch1)
    cbase = jnp.where(c == 0, s * nch0, NS * nch0 + s * nch1)

    zeros16 = jnp.zeros((LANES,), jnp.float32)

    def zrow(i, carry):
      for j in range(d // LANES):
        rowbuf[0, i, pl.ds(j * LANES, LANES)] = zeros16
      return carry

    lax.fori_loop(0, C, zrow, 0)
    row0 = s * rows_per_tile
    for kk in range(full):
      pltpu.sync_copy(rowbuf.at[0], acc.at[pl.ds(row0 + kk * C, C)])
    if rem:
      pltpu.sync_copy(rowbuf.at[0, pl.ds(0, rem)],
                      acc.at[pl.ds(row0 + full * C, rem)])
    plsc.subcore_barrier()

    def issue_idx(k, islot):
      pltpu.async_copy(eidx_hbm.at[cbase + k], iring.at[islot],
                       isem.at[islot])
      pltpu.async_copy(ew_hbm.at[cbase + k], wring.at[islot], wsem.at[islot])

    def wait_idx(k, islot):
      pltpu.make_async_copy(eidx_hbm.at[cbase + k], iring.at[islot],
                            isem.at[islot]).wait()
      pltpu.make_async_copy(ew_hbm.at[cbase + k], wring.at[islot],
                            wsem.at[islot]).wait()

    def issue_gather(k, islot, b):
      del k
      pltpu.async_copy(table_hbm.at[iring.at[islot, 0]], rowbuf.at[b],
                       gsem.at[b])

    def wait_gather(islot, b):
      pltpu.make_async_copy(table_hbm.at[iring.at[islot, 0]], rowbuf.at[b],
                            gsem.at[b]).wait()

    def issue_scatter(islot, b):
      pltpu.async_copy(rowbuf.at[b], acc.at[iring.at[islot, 1]], ssem.at[b],
                       add=True)

    def wait_scatter(islot, b):
      pltpu.make_async_copy(rowbuf.at[b], acc.at[iring.at[islot, 1]],
                            ssem.at[b]).wait()

    def scale_chunk(islot, b):
      def scale_group(g, carry2):
        wv = wring[islot, 0, pl.ds(g * LANES, LANES)]
        for eu in range(LANES):
          e = g * LANES + eu
          wb = jnp.full((LANES,), wv[eu], jnp.float32)
          for j in range(d // LANES):
            sl = pl.ds(j * LANES, LANES)
            rowbuf[b, e, sl] = rowbuf[b, e, sl] * wb
        return carry2

      lax.fori_loop(0, C // LANES, scale_group, 0)

    issue_idx(0, 0)
    issue_idx(1, 1)
    wait_idx(0, 0)
    issue_gather(0, 0, 0)

    def step(k, u):
      islot = u % IBUF
      b = u % NBUF

      @pl.when(k >= 2)
      def _():
        wait_scatter((u - 2) % IBUF, (u - 2) % NBUF)

      @pl.when(k + 1 < nch)
      def _():
        wait_idx(k + 1, (u + 1) % IBUF)
        issue_gather(k + 1, (u + 1) % IBUF, (u + 1) % NBUF)

      @pl.when(k + 2 < nch)
      def _():
        issue_idx(k + 2, (u + 2) % IBUF)

      wait_gather(islot, b)
      scale_chunk(islot, b)
      issue_scatter(islot, b)

    def main(k0, carry):
      for u in range(PERIOD):
        step(k0 * PERIOD + u, u)
      return carry

    lax.fori_loop(0, nch // PERIOD, main, 0)
    wait_scatter((PERIOD - 2) % IBUF, (PERIOD - 2) % NBUF)
    wait_scatter((PERIOD - 1) % IBUF, (PERIOD - 1) % NBUF)
    plsc.subcore_barrier()

    pltpu.sync_copy(acc.at[pl.ds(row0, rows_per_tile)],
                    out_hbm.at[c, pl.ds(row0, rows_per_tile)])

  return agg


def _mm_relu(p, w, b):
  _, n, d = p.shape
  h = w.shape[1]
  bn = 1264

  def body(p_ref, w_ref, b_ref, o_ref):
    ps = p_ref[0] + p_ref[1]
    o_ref[...] = jnp.maximum(
        jnp.dot(ps, w_ref[...], preferred_element_type=jnp.float32)
        + b_ref[...], 0.0)

  return pl.pallas_call(
      body,
      grid=(n // bn,),
      in_specs=[
          pl.BlockSpec((2, bn, d), lambda i: (0, i, 0)),
          pl.BlockSpec((d, h), lambda i: (0, 0)),
          pl.BlockSpec((1, h), lambda i: (0, 0)),
      ],
      out_specs=pl.BlockSpec((bn, h), lambda i: (i, 0)),
      out_shape=jax.ShapeDtypeStruct((n, h), jnp.float32),
  )(p, w, b.reshape(1, -1))


def _readout(hfin, wp, bp, n):
  d = hfin.shape[1]
  outd = wp.shape[1]
  bn = 1000
  nb = n // bn

  def body(h_ref, wpt_ref, wpb_ref, bp_ref, o_ref):
    i = pl.program_id(0)

    @pl.when(i == 0)
    def _():
      o_ref[...] = jnp.zeros_like(o_ref)
      o_ref[0:1, :] = bp_ref[...]

    hb = h_ref[...]
    m = jnp.max(hb, axis=1, keepdims=True)
    sm = jnp.sum(hb, axis=1, keepdims=True)
    dn = (((0,), (0,)), ((), ()))
    contrib = (lax.dot_general(m, wpt_ref[...], dn,
                               preferred_element_type=jnp.float32)
               + lax.dot_general(sm, wpb_ref[...], dn,
                                 preferred_element_type=jnp.float32))
    o_ref[0:1, :] += contrib

  out = pl.pallas_call(
      body,
      grid=(nb,),
      in_specs=[
          pl.BlockSpec((bn, d), lambda i: (i, 0)),
          pl.BlockSpec((bn, outd), lambda i: (i, 0)),
          pl.BlockSpec((bn, outd), lambda i: (nb + i, 0)),
          pl.BlockSpec((1, outd), lambda i: (0, 0)),
      ],
      out_specs=pl.BlockSpec((8, outd), lambda i: (0, 0)),
      out_shape=jax.ShapeDtypeStruct((8, outd), jnp.float32),
  )(hfin, wp, wp, bp.reshape(1, -1))
  return out[0]


def kernel(x, edge_index, edge_weight, W0, b0, W1, b1, Wp, bp):
  n, d = x.shape
  e = edge_index.shape[1]
  ncht = -(-e // (NS * C))
  ncht = -(-ncht // (2 * PERIOD)) * (2 * PERIOD)
  nch0 = int(round(ncht * SPLIT_FRAC0 / PERIOD)) * PERIOD
  nch0 = min(max(nch0, 2 * PERIOD), ncht - 2 * PERIOD)
  nch1 = ncht - nch0
  npad = -(-n // (NS * 8)) * (NS * 8)
  epad = NS * ncht * C

  src = edge_index[0]
  dst = edge_index[1]
  pad = epad - e
  srcp = jnp.concatenate([src, jnp.zeros((pad,), jnp.int32)]).reshape(-1, C)
  dstp = jnp.concatenate([dst, jnp.zeros((pad,), jnp.int32)]).reshape(-1, C)
  eidx = jnp.stack([srcp, dstp], axis=1)
  ew = jnp.concatenate(
      [edge_weight, jnp.zeros((pad,), jnp.float32)]).reshape(-1, 1, C)

  agg = _make_agg(npad, d, nch0, nch1)
  p0 = agg(x, eidx, ew)
  h1 = _mm_relu(p0, W0, b0)
  p1 = agg(h1, eidx, ew)
  h2 = _mm_relu(p1, W1, b1)
  return _readout(h2, Wp, bp, n)


# speedup vs baseline: 1.0832x; 1.0832x over previous
import functools

import jax
import jax.numpy as jnp
from jax import lax
from jax.experimental import pallas as pl
from jax.experimental.pallas import tpu as pltpu
from jax.experimental.pallas import tpu_sc as plsc

NC = 2
NS = 16
NW = NC * NS
LANES = 16
C = 112
NBUF = 3
IBUF = 6
PERIOD = 6
SPLIT_FRAC0 = 0.8


def _make_agg(npad, d, nch0, nch1):
  rows_per_tile = npad // NS
  full, rem = divmod(rows_per_tile, C)
  assert nch0 % PERIOD == 0 and nch1 % PERIOD == 0
  mesh = plsc.VectorSubcoreMesh(core_axis_name="c", subcore_axis_name="s")

  @functools.partial(
      pl.kernel,
      out_type=jax.ShapeDtypeStruct((NC, npad, d), jnp.float32),
      mesh=mesh,
      scratch_types=[
          pltpu.VMEM((IBUF, 2, C), jnp.int32),
          pltpu.VMEM((IBUF, 1, C), jnp.float32),
          pltpu.VMEM((NBUF, C, d), jnp.float32),
          pltpu.VMEM_SHARED((npad, d), jnp.float32),
          pltpu.SemaphoreType.DMA((IBUF,)),
          pltpu.SemaphoreType.DMA((IBUF,)),
          pltpu.SemaphoreType.DMA((NBUF,)),
          pltpu.SemaphoreType.DMA((NBUF,)),
      ],
  )
  def agg(table_hbm, eidx_hbm, ew_hbm, out_hbm,
          iring, wring, rowbuf, acc, isem, wsem, gsem, ssem):
    c = lax.axis_index("c")
    s = lax.axis_index("s")
    nch = jnp.where(c == 0, nch0, nch1)
    cbase = jnp.where(c == 0, s * nch0, NS * nch0 + s * nch1)

    zeros16 = jnp.zeros((LANES,), jnp.float32)

    def zrow(i, carry):
      for j in range(d // LANES):
        rowbuf[0, i, pl.ds(j * LANES, LANES)] = zeros16
      return carry

    lax.fori_loop(0, C, zrow, 0)
    row0 = s * rows_per_tile
    for kk in range(full):
      pltpu.sync_copy(rowbuf.at[0], acc.at[pl.ds(row0 + kk * C, C)])
    if rem:
      pltpu.sync_copy(rowbuf.at[0, pl.ds(0, rem)],
                      acc.at[pl.ds(row0 + full * C, rem)])
    plsc.subcore_barrier()

    def issue_idx(k, islot):
      pltpu.async_copy(eidx_hbm.at[cbase + k], iring.at[islot],
                       isem.at[islot])
      pltpu.async_copy(ew_hbm.at[cbase + k], wring.at[islot], wsem.at[islot])

    def wait_idx(k, islot):
      pltpu.make_async_copy(eidx_hbm.at[cbase + k], iring.at[islot],
                            isem.at[islot]).wait()
      pltpu.make_async_copy(ew_hbm.at[cbase + k], wring.at[islot],
                            wsem.at[islot]).wait()

    def issue_gather(k, islot, b):
      del k
      pltpu.async_copy(table_hbm.at[iring.at[islot, 0]], rowbuf.at[b],
                       gsem.at[b])

    def wait_gather(islot, b):
      pltpu.make_async_copy(table_hbm.at[iring.at[islot, 0]], rowbuf.at[b],
                            gsem.at[b]).wait()

    def issue_scatter(islot, b):
      pltpu.async_copy(rowbuf.at[b], acc.at[iring.at[islot, 1]], ssem.at[b],
                       add=True)

    def wait_scatter(islot, b):
      pltpu.make_async_copy(rowbuf.at[b], acc.at[iring.at[islot, 1]],
                            ssem.at[b]).wait()

    def scale_chunk(islot, b):
      def scale_group(g, carry2):
        wv = wring[islot, 0, pl.ds(g * LANES, LANES)]
        for eu in range(LANES):
          e = g * LANES + eu
          wb = jnp.full((LANES,), wv[eu], jnp.float32)
          for j in range(d // LANES):
            sl = pl.ds(j * LANES, LANES)
            rowbuf[b, e, sl] = rowbuf[b, e, sl] * wb
        return carry2

      lax.fori_loop(0, C // LANES, scale_group, 0)

    issue_idx(0, 0)
    issue_idx(1, 1)
    wait_idx(0, 0)
    issue_gather(0, 0, 0)

    def step(k, u):
      islot = u % IBUF
      b = u % NBUF

      @pl.when(k >= 2)
      def _():
        wait_scatter((u - 2) % IBUF, (u - 2) % NBUF)

      @pl.when(k + 1 < nch)
      def _():
        wait_idx(k + 1, (u + 1) % IBUF)
        issue_gather(k + 1, (u + 1) % IBUF, (u + 1) % NBUF)

      @pl.when(k + 2 < nch)
      def _():
        issue_idx(k + 2, (u + 2) % IBUF)

      wait_gather(islot, b)
      scale_chunk(islot, b)
      issue_scatter(islot, b)

    def main(k0, carry):
      for u in range(PERIOD):
        step(k0 * PERIOD + u, u)
      return carry

    lax.fori_loop(0, nch // PERIOD, main, 0)
    wait_scatter((PERIOD - 2) % IBUF, (PERIOD - 2) % NBUF)
    wait_scatter((PERIOD - 1) % IBUF, (PERIOD - 1) % NBUF)
    plsc.subcore_barrier()

    pltpu.sync_copy(acc.at[pl.ds(row0, rows_per_tile)],
                    out_hbm.at[c, pl.ds(row0, rows_per_tile)])

  return agg


def _mm_relu(p, w, b, out_dtype=jnp.float32):
  _, n, d = p.shape
  h = w.shape[1]
  bn = 1264

  def body(p_ref, w_ref, b_ref, o_ref):
    ps = p_ref[0] + p_ref[1]
    o_ref[...] = jnp.maximum(
        jnp.dot(ps, w_ref[...], preferred_element_type=jnp.float32)
        + b_ref[...], 0.0).astype(o_ref.dtype)

  return pl.pallas_call(
      body,
      grid=(n // bn,),
      in_specs=[
          pl.BlockSpec((2, bn, d), lambda i: (0, i, 0)),
          pl.BlockSpec((d, h), lambda i: (0, 0)),
          pl.BlockSpec((1, h), lambda i: (0, 0)),
      ],
      out_specs=pl.BlockSpec((bn, h), lambda i: (i, 0)),
      out_shape=jax.ShapeDtypeStruct((n, h), out_dtype),
  )(p, w, b.reshape(1, -1))


def _readout(hfin, wp, bp, n):
  d = hfin.shape[1]
  outd = wp.shape[1]
  bn = 1000
  nb = n // bn

  def body(h_ref, wpt_ref, wpb_ref, bp_ref, o_ref):
    i = pl.program_id(0)

    @pl.when(i == 0)
    def _():
      o_ref[...] = jnp.zeros_like(o_ref)
      o_ref[0:1, :] = bp_ref[...]

    hb = h_ref[...]
    m = jnp.max(hb, axis=1, keepdims=True)
    sm = jnp.sum(hb, axis=1, keepdims=True)
    dn = (((0,), (0,)), ((), ()))
    contrib = (lax.dot_general(m, wpt_ref[...], dn,
                               preferred_element_type=jnp.float32)
               + lax.dot_general(sm, wpb_ref[...], dn,
                                 preferred_element_type=jnp.float32))
    o_ref[0:1, :] += contrib

  out = pl.pallas_call(
      body,
      grid=(nb,),
      in_specs=[
          pl.BlockSpec((bn, d), lambda i: (i, 0)),
          pl.BlockSpec((bn, outd), lambda i: (i, 0)),
          pl.BlockSpec((bn, outd), lambda i: (nb + i, 0)),
          pl.BlockSpec((1, outd), lambda i: (0, 0)),
      ],
      out_specs=pl.BlockSpec((8, outd), lambda i: (0, 0)),
      out_shape=jax.ShapeDtypeStruct((8, outd), jnp.float32),
  )(hfin, wp, wp, bp.reshape(1, -1))
  return out[0]


def kernel(x, edge_index, edge_weight, W0, b0, W1, b1, Wp, bp):
  n, d = x.shape
  e = edge_index.shape[1]
  ncht = -(-e // (NS * C))
  ncht = -(-ncht // (2 * PERIOD)) * (2 * PERIOD)
  nch0 = int(round(ncht * SPLIT_FRAC0 / PERIOD)) * PERIOD
  nch0 = min(max(nch0, 2 * PERIOD), ncht - 2 * PERIOD)
  nch1 = ncht - nch0
  npad = -(-n // (NS * 8)) * (NS * 8)
  epad = NS * ncht * C

  src = edge_index[0]
  dst = edge_index[1]
  pad = epad - e
  srcp = jnp.concatenate([src, jnp.zeros((pad,), jnp.int32)]).reshape(-1, C)
  dstp = jnp.concatenate([dst, jnp.zeros((pad,), jnp.int32)]).reshape(-1, C)
  eidx = jnp.stack([srcp, dstp], axis=1)
  ew = jnp.concatenate(
      [edge_weight, jnp.zeros((pad,), jnp.float32)]).reshape(-1, 1, C)

  agg = _make_agg(npad, d, nch0, nch1)
  p0 = agg(x, eidx, ew)
  h1 = _mm_relu(p0, W0, b0)
  p1 = agg(h1, eidx, ew)
  h2 = _mm_relu(p1, W1, b1)
  return _readout(h2, Wp, bp, n)
